# baseline (speedup 1.0000x reference)
import functools
import jax
import jax.numpy as jnp
from jax import lax
from jax.experimental import pallas as pl
from jax.experimental.pallas import tpu as pltpu
from jax.experimental.pallas import tpu_sc as plsc

_NC, _NS = 2, 16
_NW = _NC * _NS
_CHUNK_ROWS = 32


def kernel(base, source):
    del base
    b, s, d = source.shape
    rows = b * s
    rows_per_w = rows // _NW
    nchunks = rows_per_w // _CHUNK_ROWS
    src2d = source.reshape(rows, d)
    mesh = plsc.VectorSubcoreMesh(core_axis_name="c", subcore_axis_name="s")

    @functools.partial(
        pl.kernel,
        mesh=mesh,
        out_type=jax.ShapeDtypeStruct((rows, d), source.dtype),
        scratch_types=[
            pltpu.VMEM((3, _CHUNK_ROWS, d), jnp.float32),
            pltpu.SemaphoreType.DMA((3,)),
            pltpu.SemaphoreType.DMA((3,)),
        ],
    )
    def _copy(src_hbm, out_hbm, buf, in_sems, out_sems):
        wid = lax.axis_index("s") * _NC + lax.axis_index("c")
        base_row = wid * rows_per_w

        def _in(g, bslot):
            return pltpu.make_async_copy(
                src_hbm.at[pl.ds(base_row + g * _CHUNK_ROWS, _CHUNK_ROWS)],
                buf.at[bslot],
                in_sems.at[bslot],
            )

        def _out(g, bslot):
            return pltpu.make_async_copy(
                buf.at[bslot],
                out_hbm.at[pl.ds(base_row + g * _CHUNK_ROWS, _CHUNK_ROWS)],
                out_sems.at[bslot],
            )

        _in(0, 0).start()
        _in(1, 1).start()
        for g in range(nchunks):
            bslot = g % 3
            _in(g, bslot).wait()
            _out(g, bslot).start()
            if g + 2 < nchunks:
                if g >= 1:
                    _out(g - 1, (g - 1) % 3).wait()
                _in(g + 2, (g + 2) % 3).start()
        _out(nchunks - 3, (nchunks - 3) % 3).wait()
        _out(nchunks - 2, (nchunks - 2) % 3).wait()
        _out(nchunks - 1, (nchunks - 1) % 3).wait()

    out = _copy(src2d)
    return out.reshape(b, s, d)


# speedup vs baseline: 1.3781x; 1.3781x over previous
import jax
import jax.numpy as jnp
from jax.experimental import pallas as pl
from jax.experimental.pallas import tpu as pltpu

_BLOCK_ROWS = 2048


def _copy_body(src_ref, out_ref):
    out_ref[...] = src_ref[...]


def kernel(base, source):
    del base
    b, s, d = source.shape
    rows = b * s
    src2d = source.reshape(rows, d)
    grid = (rows // _BLOCK_ROWS,)
    out = pl.pallas_call(
        _copy_body,
        out_shape=jax.ShapeDtypeStruct((rows, d), source.dtype),
        grid=grid,
        in_specs=[pl.BlockSpec((_BLOCK_ROWS, d), lambda i: (i, 0))],
        out_specs=pl.BlockSpec((_BLOCK_ROWS, d), lambda i: (i, 0)),
        compiler_params=pltpu.CompilerParams(
            dimension_semantics=("parallel",),
        ),
    )(src2d)
    return out.reshape(b, s, d)
